# baseline (speedup 1.0000x reference)
import jax
import jax.numpy as jnp
from jax.experimental import pallas as pl
from jax.experimental.pallas import tpu as pltpu

_N = 4096
_C = 32
_BLOCK = 512
_NB = _N // _BLOCK


def _dot(a, b):
    return jax.lax.dot_general(
        a, b, (((1,), (0,)), ((), ())),
        preferred_element_type=jnp.float32,
    )


def _snn_kernel(lap_ref, x_ref, w_ref, out_ref, u_ref):
    i = pl.program_id(0)
    h = pl.program_id(1)
    ib = pl.program_id(2)
    slab = lap_ref[0]
    sl = pl.ds(ib * _BLOCK, _BLOCK)

    @pl.when(jnp.logical_and(i == 0, h == 0))
    def _():
        out_ref[0, sl, :] = _dot(x_ref[0, sl, :], w_ref[0])

    @pl.when(h == 0)
    def _():
        u_blk = _dot(slab, x_ref[0])
        u_ref[sl, :] = u_blk
        w_u = w_ref[pl.ds(1 + 2 * i, 1)][0]
        out_ref[0, sl, :] += _dot(u_blk, w_u)

    @pl.when(h == 1)
    def _():
        v_blk = _dot(slab, u_ref[:, :])
        w_v = w_ref[pl.ds(2 + 2 * i, 1)][0]
        out_ref[0, sl, :] += _dot(v_blk, w_v)


def kernel(x_1, laplacian_all, weight_1):
    w = jnp.transpose(weight_1, (2, 0, 1))
    out = pl.pallas_call(
        _snn_kernel,
        grid=(2, 2, _NB),
        in_specs=[
            pl.BlockSpec((1, _BLOCK, _N), lambda i, h, ib: (i, ib, 0)),
            pl.BlockSpec((1, _N, _C), lambda i, h, ib: (0, 0, 0)),
            pl.BlockSpec((5, _C, _C), lambda i, h, ib: (0, 0, 0)),
        ],
        out_specs=pl.BlockSpec((1, _N, _C), lambda i, h, ib: (0, 0, 0)),
        out_shape=jax.ShapeDtypeStruct((1, _N, _C), jnp.float32),
        scratch_shapes=[pltpu.VMEM((_N, _C), jnp.float32)],
    )(laplacian_all, x_1, w)
    return out


# speedup vs baseline: 1.0716x; 1.0716x over previous
import jax
import jax.numpy as jnp
from jax.experimental import pallas as pl
from jax.experimental.pallas import tpu as pltpu

_N = 4096
_C = 32
_BLOCK = 512
_NB = _N // _BLOCK

_DN_T = (((1,), (1,)), ((), ()))
_DN_STD = (((1,), (0,)), ((), ()))


def _snn_kernel(lap_ref, xt_ref, wt_ref, out_ref, ut_ref):
    i = pl.program_id(0)
    h = pl.program_id(1)
    ib = pl.program_id(2)
    slab = lap_ref[0]
    sl = pl.ds(ib * _BLOCK, _BLOCK)

    @pl.when(jnp.logical_and(jnp.logical_and(i == 0, h == 0), ib == 0))
    def _():
        out_ref[:, :] = jax.lax.dot_general(
            wt_ref[0], xt_ref[:, :], _DN_STD,
            preferred_element_type=jnp.float32)

    @pl.when(h == 0)
    def _():
        u_blk = jax.lax.dot_general(
            xt_ref[:, :], slab, _DN_T, preferred_element_type=jnp.float32)
        ut_ref[:, sl] = u_blk
        wt_u = wt_ref[pl.ds(1 + 2 * i, 1)][0]
        out_ref[:, sl] += jax.lax.dot_general(
            wt_u, u_blk, _DN_STD, preferred_element_type=jnp.float32)

    @pl.when(h == 1)
    def _():
        v_blk = jax.lax.dot_general(
            ut_ref[:, :], slab, _DN_T, preferred_element_type=jnp.float32)
        wt_v = wt_ref[pl.ds(2 + 2 * i, 1)][0]
        out_ref[:, sl] += jax.lax.dot_general(
            wt_v, v_blk, _DN_STD, preferred_element_type=jnp.float32)


def kernel(x_1, laplacian_all, weight_1):
    xt = jnp.transpose(x_1[0])
    wt = jnp.transpose(weight_1, (2, 1, 0))
    yt = pl.pallas_call(
        _snn_kernel,
        grid=(2, 2, _NB),
        in_specs=[
            pl.BlockSpec((1, _BLOCK, _N), lambda i, h, ib: (i, ib, 0)),
            pl.BlockSpec((_C, _N), lambda i, h, ib: (0, 0)),
            pl.BlockSpec((5, _C, _C), lambda i, h, ib: (0, 0, 0)),
        ],
        out_specs=pl.BlockSpec((_C, _N), lambda i, h, ib: (0, 0)),
        out_shape=jax.ShapeDtypeStruct((_C, _N), jnp.float32),
        scratch_shapes=[pltpu.VMEM((_C, _N), jnp.float32)],
    )(laplacian_all, xt, wt)
    return jnp.transpose(yt)[None]
